# baseline (device time: 25423 ns/iter reference)
import jax
import jax.numpy as jnp
from jax import lax
from jax.experimental import pallas as pl
from jax.experimental.pallas import tpu as pltpu

N_DEV = 4
N_CHUNK = 4


def kernel(x, Win0, Wout0, Win1, Wout1, Win2, Wout2):
    b, d = x.shape
    k_in, hsh = Win0.shape
    ck = d // N_CHUNK
    rows = b // N_DEV

    def body(
        x_ref,
        win0_hbm,
        wout0_hbm,
        win1_hbm,
        wout1_hbm,
        win2_hbm,
        wout2_hbm,
        out_ref,
        win_v,
        wout_v,
        psend_ref,
        comm_ref,
        p2_ref,
        rs2_ref,
        w_sems,
        send_sems,
        recv_sems,
        rs2_send_sems,
        rs2_recv_sems,
    ):
        my = lax.axis_index("i")

        loads = [
            pltpu.make_async_copy(win0_hbm, win_v.at[0], w_sems.at[0]),
            pltpu.make_async_copy(wout0_hbm, wout_v.at[0], w_sems.at[1]),
            pltpu.make_async_copy(win1_hbm, win_v.at[1], w_sems.at[2]),
            pltpu.make_async_copy(wout1_hbm, wout_v.at[1], w_sems.at[3]),
            pltpu.make_async_copy(win2_hbm, win_v.at[2], w_sems.at[4]),
            pltpu.make_async_copy(wout2_hbm, wout_v.at[2], w_sems.at[5]),
        ]
        loads[0].start()
        loads[1].start()

        barrier_sem = pltpu.get_barrier_semaphore()
        for idx in range(1, N_DEV):
            pl.semaphore_signal(
                barrier_sem,
                inc=1,
                device_id=((my + idx) % N_DEV,),
                device_id_type=pl.DeviceIdType.MESH,
            )

        sends = []

        loads[0].wait()
        h = jnp.maximum(
            jnp.dot(x_ref[:, :], win_v[0], preferred_element_type=jnp.float32),
            0.0,
        )
        pl.semaphore_wait(barrier_sem, N_DEV - 1)

        for l in range(2):
            loads[2 * l + 2].start()
            loads[2 * l + 1].wait()
            for c in range(N_CHUNK):
                pc = jnp.dot(
                    h,
                    wout_v[l][:, c * ck : (c + 1) * ck],
                    preferred_element_type=jnp.float32,
                )
                psend_ref[l, c, :, :] = pc.astype(jnp.bfloat16)
                for idx in (2, 1, 3):
                    rdma = pltpu.make_async_remote_copy(
                        src_ref=psend_ref.at[l, c],
                        dst_ref=comm_ref.at[l, c, N_DEV - 1 - idx],
                        send_sem=send_sems.at[l, c, idx - 1],
                        recv_sem=recv_sems.at[l, c, N_DEV - 1 - idx],
                        device_id=((my + idx) % N_DEV,),
                        device_id_type=pl.DeviceIdType.MESH,
                    )
                    rdma.start()
                    sends.append(rdma)

            loads[2 * l + 3].start()
            loads[2 * l + 2].wait()
            acc_h = jnp.zeros((b, hsh), jnp.float32)
            for c in range(N_CHUNK):
                xc = psend_ref[l, c, :, :].astype(jnp.float32)
                for j in range(N_DEV - 1):
                    recv = pltpu.make_async_remote_copy(
                        src_ref=comm_ref.at[l, c, j],
                        dst_ref=comm_ref.at[l, c, j],
                        send_sem=recv_sems.at[l, c, j],
                        recv_sem=recv_sems.at[l, c, j],
                        device_id=(my,),
                        device_id_type=pl.DeviceIdType.MESH,
                    )
                    recv.wait_recv()
                    xc = xc + comm_ref[l, c, j, :, :].astype(jnp.float32)
                acc_h = acc_h + jnp.dot(
                    xc,
                    win_v[l + 1][c * ck : (c + 1) * ck, :],
                    preferred_element_type=jnp.float32,
                )
            h = jnp.maximum(acc_h, 0.0)

        loads[5].wait()
        p2_ref[:, :] = jnp.dot(
            h, wout_v[2], preferred_element_type=jnp.float32
        ).astype(jnp.bfloat16)
        for idx in (2, 1, 3):
            peer = (my + idx) % N_DEV
            rdma = pltpu.make_async_remote_copy(
                src_ref=p2_ref.at[pl.ds(peer * rows, rows)],
                dst_ref=rs2_ref.at[N_DEV - 1 - idx],
                send_sem=rs2_send_sems.at[idx - 1],
                recv_sem=rs2_recv_sems.at[N_DEV - 1 - idx],
                device_id=(peer,),
                device_id_type=pl.DeviceIdType.MESH,
            )
            rdma.start()
            sends.append(rdma)
        own = p2_ref[pl.ds(my * rows, rows), :].astype(jnp.float32)
        for j in range(N_DEV - 1):
            recv = pltpu.make_async_remote_copy(
                src_ref=rs2_ref.at[j],
                dst_ref=rs2_ref.at[j],
                send_sem=rs2_recv_sems.at[j],
                recv_sem=rs2_recv_sems.at[j],
                device_id=(my,),
                device_id_type=pl.DeviceIdType.MESH,
            )
            recv.wait_recv()
            own = own + rs2_ref[j, :, :].astype(jnp.float32)
        out_ref[:, :] = own

        for rdma in sends:
            rdma.wait_send()

    return pl.pallas_call(
        body,
        out_shape=jax.ShapeDtypeStruct((rows, d), jnp.float32),
        in_specs=[pl.BlockSpec(memory_space=pltpu.VMEM)]
        + [pl.BlockSpec(memory_space=pl.ANY)] * 6,
        out_specs=pl.BlockSpec(memory_space=pltpu.VMEM),
        scratch_shapes=[
            pltpu.VMEM((3, k_in, hsh), jnp.float32),
            pltpu.VMEM((3, hsh, d), jnp.float32),
            pltpu.VMEM((2, N_CHUNK, b, ck), jnp.bfloat16),
            pltpu.VMEM((2, N_CHUNK, N_DEV - 1, b, ck), jnp.bfloat16),
            pltpu.VMEM((b, d), jnp.bfloat16),
            pltpu.VMEM((N_DEV - 1, rows, d), jnp.bfloat16),
            pltpu.SemaphoreType.DMA((6,)),
            pltpu.SemaphoreType.DMA((2, N_CHUNK, N_DEV - 1)),
            pltpu.SemaphoreType.DMA((2, N_CHUNK, N_DEV - 1)),
            pltpu.SemaphoreType.DMA((N_DEV - 1,)),
            pltpu.SemaphoreType.DMA((N_DEV - 1,)),
        ],
        compiler_params=pltpu.CompilerParams(collective_id=0),
    )(x, Win0, Wout0, Win1, Wout1, Win2, Wout2)


# device time: 24567 ns/iter; 1.0348x vs baseline; 1.0348x over previous
import jax
import jax.numpy as jnp
from jax import lax
from jax.experimental import pallas as pl
from jax.experimental.pallas import tpu as pltpu

N_DEV = 4
N_CHUNK = 4


def kernel(x, Win0, Wout0, Win1, Wout1, Win2, Wout2):
    b, d = x.shape
    ck = d // N_CHUNK
    rows = b // N_DEV

    def body(
        x_ref,
        win0_ref,
        wout0_ref,
        win1_ref,
        wout1_ref,
        win2_ref,
        wout2_ref,
        out_ref,
        psend_ref,
        comm_ref,
        p2_ref,
        rs2_ref,
        send_sems,
        recv_sems,
        rs2_send_sems,
        rs2_recv_sems,
    ):
        my = lax.axis_index("i")

        barrier_sem = pltpu.get_barrier_semaphore()
        for idx in range(1, N_DEV):
            pl.semaphore_signal(
                barrier_sem,
                inc=1,
                device_id=((my + idx) % N_DEV,),
                device_id_type=pl.DeviceIdType.MESH,
            )

        wins = [win0_ref, win1_ref, win2_ref]
        wouts = [wout0_ref, wout1_ref, wout2_ref]
        sends = []

        h = jnp.maximum(
            jnp.dot(x_ref[:, :], win0_ref[:, :], preferred_element_type=jnp.float32),
            0.0,
        )
        pl.semaphore_wait(barrier_sem, N_DEV - 1)

        for l in range(2):
            win_next = wins[l + 1]
            for c in range(N_CHUNK):
                pc = jnp.dot(
                    h,
                    wouts[l][:, c * ck : (c + 1) * ck],
                    preferred_element_type=jnp.float32,
                )
                psend_ref[l, c, :, :] = pc.astype(jnp.bfloat16)
                for idx in (2, 1, 3):
                    rdma = pltpu.make_async_remote_copy(
                        src_ref=psend_ref.at[l, c],
                        dst_ref=comm_ref.at[l, c, N_DEV - 1 - idx],
                        send_sem=send_sems.at[l, c, idx - 1],
                        recv_sem=recv_sems.at[l, c, N_DEV - 1 - idx],
                        device_id=((my + idx) % N_DEV,),
                        device_id_type=pl.DeviceIdType.MESH,
                    )
                    rdma.start()
                    sends.append(rdma)

            acc_h = jnp.zeros((b, win_next.shape[1]), jnp.float32)
            for c in range(N_CHUNK):
                xc = psend_ref[l, c, :, :].astype(jnp.float32)
                for j in range(N_DEV - 1):
                    recv = pltpu.make_async_remote_copy(
                        src_ref=comm_ref.at[l, c, j],
                        dst_ref=comm_ref.at[l, c, j],
                        send_sem=recv_sems.at[l, c, j],
                        recv_sem=recv_sems.at[l, c, j],
                        device_id=(my,),
                        device_id_type=pl.DeviceIdType.MESH,
                    )
                    recv.wait_recv()
                    xc = xc + comm_ref[l, c, j, :, :].astype(jnp.float32)
                acc_h = acc_h + jnp.dot(
                    xc,
                    win_next[c * ck : (c + 1) * ck, :],
                    preferred_element_type=jnp.float32,
                )
            h = jnp.maximum(acc_h, 0.0)

        p2_ref[:, :] = jnp.dot(
            h, wout2_ref[:, :], preferred_element_type=jnp.float32
        ).astype(jnp.bfloat16)
        for idx in (2, 1, 3):
            peer = (my + idx) % N_DEV
            rdma = pltpu.make_async_remote_copy(
                src_ref=p2_ref.at[pl.ds(peer * rows, rows)],
                dst_ref=rs2_ref.at[N_DEV - 1 - idx],
                send_sem=rs2_send_sems.at[idx - 1],
                recv_sem=rs2_recv_sems.at[N_DEV - 1 - idx],
                device_id=(peer,),
                device_id_type=pl.DeviceIdType.MESH,
            )
            rdma.start()
            sends.append(rdma)
        own = p2_ref[pl.ds(my * rows, rows), :].astype(jnp.float32)
        for j in range(N_DEV - 1):
            recv = pltpu.make_async_remote_copy(
                src_ref=rs2_ref.at[j],
                dst_ref=rs2_ref.at[j],
                send_sem=rs2_recv_sems.at[j],
                recv_sem=rs2_recv_sems.at[j],
                device_id=(my,),
                device_id_type=pl.DeviceIdType.MESH,
            )
            recv.wait_recv()
            own = own + rs2_ref[j, :, :].astype(jnp.float32)
        out_ref[:, :] = own

        for rdma in sends:
            rdma.wait_send()

    return pl.pallas_call(
        body,
        out_shape=jax.ShapeDtypeStruct((rows, d), jnp.float32),
        in_specs=[pl.BlockSpec(memory_space=pltpu.VMEM)] * 7,
        out_specs=pl.BlockSpec(memory_space=pltpu.VMEM),
        scratch_shapes=[
            pltpu.VMEM((2, N_CHUNK, b, ck), jnp.bfloat16),
            pltpu.VMEM((2, N_CHUNK, N_DEV - 1, b, ck), jnp.bfloat16),
            pltpu.VMEM((b, d), jnp.bfloat16),
            pltpu.VMEM((N_DEV - 1, rows, d), jnp.bfloat16),
            pltpu.SemaphoreType.DMA((2, N_CHUNK, N_DEV - 1)),
            pltpu.SemaphoreType.DMA((2, N_CHUNK, N_DEV - 1)),
            pltpu.SemaphoreType.DMA((N_DEV - 1,)),
            pltpu.SemaphoreType.DMA((N_DEV - 1,)),
        ],
        compiler_params=pltpu.CompilerParams(collective_id=0),
    )(x, Win0, Wout0, Win1, Wout1, Win2, Wout2)
